# Initial kernel scaffold; baseline (speedup 1.0000x reference)
import jax
import jax.numpy as jnp
from jax.experimental import pallas as pl


def kernel(x, edge_index, edge_attr, W_gcn, b_gcn, W_ih, W_hh, b_ih, b_hh, W_lin, b_lin):
    raise NotImplementedError("write your pallas kernel here")


# speedup vs baseline: 1.5400x; 1.5400x over previous
import functools

import jax
import jax.numpy as jnp
from jax.experimental import pallas as pl
from jax.experimental.pallas import tpu as pltpu


def _pick_block(e):
    for cand in (1024, 1000, 800, 640, 512, 400, 320, 256, 200, 160, 128, 64, 32, 16, 8):
        if e % cand == 0:
            return cand
    return e


def _lstm_body(ssrc_ref, sdst_ref, attr_ref, wgcn_ref, bgcn_ref, wsrc_ref,
               wdst_ref, wattr_ref, bias_ref, whh_ref, wlin_ref, blin_ref,
               out_ref, h_scr, c_scr, hs_scr, pre_scr, *, be, lh):
    pi = pl.program_id(0)

    @pl.when(pi == 0)
    def _init():
        h_scr[...] = jnp.zeros_like(h_scr)
        c_scr[...] = jnp.zeros_like(c_scr)

    wgcn = wgcn_ref[...]
    bgcn = bgcn_ref[...]
    fs = jnp.maximum(ssrc_ref[...] * wgcn + bgcn, 0.0)
    fd = jnp.maximum(sdst_ref[...] * wgcn + bgcn, 0.0)
    pre = (jnp.dot(fs, wsrc_ref[...], preferred_element_type=jnp.float32)
           + jnp.dot(fd, wdst_ref[...], preferred_element_type=jnp.float32)
           + jnp.dot(attr_ref[...], wattr_ref[...],
                     preferred_element_type=jnp.float32)
           + bias_ref[...])
    pre_scr[...] = pre

    whh = whh_ref[...]

    def step(t, carry):
        h, c = carry
        gates = pre_scr[pl.ds(t, 1), :] + jnp.dot(
            h, whh, preferred_element_type=jnp.float32)
        ii = 1.0 / (1.0 + jnp.exp(-gates[:, 0:lh]))
        ff = 1.0 / (1.0 + jnp.exp(-gates[:, lh:2 * lh]))
        gg = jnp.tanh(gates[:, 2 * lh:3 * lh])
        oo = 1.0 / (1.0 + jnp.exp(-gates[:, 3 * lh:4 * lh]))
        c2 = ff * c + ii * gg
        h2 = oo * jnp.tanh(c2)
        hs_scr[pl.ds(t, 1), :] = h2
        return (h2, c2)

    hN, cN = jax.lax.fori_loop(0, be, step, (h_scr[...], c_scr[...]))
    h_scr[...] = hN
    c_scr[...] = cN
    out_ref[...] = jnp.dot(hs_scr[...], wlin_ref[...],
                           preferred_element_type=jnp.float32) + blin_ref[...]


def _edge_lstm(s_src, s_dst, attr_p, W_gcn, b_gcn, Wsrc_T, Wdst_T, Wattr_T,
               bias, Whh_T, Wlin_T, blin):
    e = s_src.shape[0]
    h = W_gcn.shape[1]
    lh = Whh_T.shape[0]
    g4 = 4 * lh
    ap = attr_p.shape[1]
    be = _pick_block(e)
    nb = e // be

    body = functools.partial(_lstm_body, be=be, lh=lh)
    out = pl.pallas_call(
        body,
        grid=(nb,),
        in_specs=[
            pl.BlockSpec((be, 1), lambda i: (i, 0)),
            pl.BlockSpec((be, 1), lambda i: (i, 0)),
            pl.BlockSpec((be, ap), lambda i: (i, 0)),
            pl.BlockSpec((1, h), lambda i: (0, 0)),
            pl.BlockSpec((1, h), lambda i: (0, 0)),
            pl.BlockSpec((h, g4), lambda i: (0, 0)),
            pl.BlockSpec((h, g4), lambda i: (0, 0)),
            pl.BlockSpec((ap, g4), lambda i: (0, 0)),
            pl.BlockSpec((1, g4), lambda i: (0, 0)),
            pl.BlockSpec((lh, g4), lambda i: (0, 0)),
            pl.BlockSpec((lh, 1), lambda i: (0, 0)),
            pl.BlockSpec((1, 1), lambda i: (0, 0)),
        ],
        out_specs=pl.BlockSpec((be, 1), lambda i: (i, 0)),
        out_shape=jax.ShapeDtypeStruct((e, 1), jnp.float32),
        scratch_shapes=[
            pltpu.VMEM((1, lh), jnp.float32),
            pltpu.VMEM((1, lh), jnp.float32),
            pltpu.VMEM((be, lh), jnp.float32),
            pltpu.VMEM((be, g4), jnp.float32),
        ],
    )(s_src, s_dst, attr_p, W_gcn, b_gcn, Wsrc_T, Wdst_T, Wattr_T, bias,
      Whh_T, Wlin_T, blin)
    return out


def kernel(x, edge_index, edge_attr, W_gcn, b_gcn, W_ih, W_hh, b_ih, b_hh,
           W_lin, b_lin):
    n = x.shape[0]
    e = edge_index.shape[1]
    h = W_gcn.shape[1]
    lh = W_hh.shape[1]
    a = edge_attr.shape[1]

    src = edge_index[0]
    dst = edge_index[1]

    xs = x[:, 0]
    deg = jnp.zeros((n,), jnp.float32).at[dst].add(1.0) + 1.0
    dinv = jax.lax.rsqrt(deg)
    av = xs * dinv
    spart = jnp.zeros((n,), jnp.float32).at[dst].add(av[src])
    s = dinv * spart + xs * dinv * dinv
    s_src = s[src][:, None]
    s_dst = s[dst][:, None]

    attr_p = jnp.pad(edge_attr, ((0, 0), (0, 8 - a)))
    Wsrc_T = W_ih[:, :h].T
    Wdst_T = W_ih[:, h:2 * h].T
    Wattr_T = jnp.pad(W_ih[:, 2 * h:].T, ((0, 8 - a), (0, 0)))
    bias = (b_ih + b_hh)[None, :]
    Whh_T = W_hh.T
    Wlin_T = W_lin.T
    blin = b_lin[None, :]

    out = _edge_lstm(s_src, s_dst, attr_p, W_gcn, b_gcn[None, :], Wsrc_T,
                     Wdst_T, Wattr_T, bias, Whh_T, Wlin_T, blin)
    return out.reshape(-1)


# speedup vs baseline: 1.6224x; 1.0535x over previous
import functools

import jax
import jax.numpy as jnp
from jax.experimental import pallas as pl
from jax.experimental.pallas import tpu as pltpu


def _pick_block(e):
    for cand in (1024, 1000, 800, 640, 512, 400, 320, 256, 200, 160, 128, 64, 32, 16, 8):
        if e % cand == 0:
            return cand
    return e


def _lstm_body(ssrc_ref, sdst_ref, attr_ref, wgcn_ref, bgcn_ref, wsrc_ref,
               wdst_ref, wattr_ref, bias_ref, whh_ref, wlin_ref, blin_ref,
               out_ref, h_scr, c_scr, hs_scr, pre_scr, *, be, lh):
    pi = pl.program_id(0)

    @pl.when(pi == 0)
    def _init():
        h_scr[...] = jnp.zeros_like(h_scr)
        c_scr[...] = jnp.zeros_like(c_scr)

    wgcn = wgcn_ref[...]
    bgcn = bgcn_ref[...]
    fs = jnp.maximum(ssrc_ref[...] * wgcn + bgcn, 0.0)
    fd = jnp.maximum(sdst_ref[...] * wgcn + bgcn, 0.0)
    pre = (jnp.dot(fs, wsrc_ref[...], preferred_element_type=jnp.float32)
           + jnp.dot(fd, wdst_ref[...], preferred_element_type=jnp.float32)
           + jnp.dot(attr_ref[...], wattr_ref[...],
                     preferred_element_type=jnp.float32)
           + bias_ref[...])
    pre_scr[...] = pre

    whh = whh_ref[...]
    lane = jax.lax.broadcasted_iota(jnp.int32, (1, 4 * lh), 1)
    is_g = (lane >= 2 * lh) & (lane < 3 * lh)
    sv = jnp.where(is_g, 1.0, 0.5)
    pa = sv
    pb = jnp.where(is_g, 0.0, 0.5)

    def chunk(k, carry):
        h, c = carry
        rows = pre_scr[pl.ds(k * 8, 8), :]
        for j in range(8):
            gates = rows[j:j + 1, :] + jnp.dot(
                h, whh, preferred_element_type=jnp.float32)
            act = jnp.tanh(gates * sv) * pa + pb
            ii = act[:, 0:lh]
            ff = act[:, lh:2 * lh]
            gg = act[:, 2 * lh:3 * lh]
            oo = act[:, 3 * lh:4 * lh]
            c = ff * c + ii * gg
            h = oo * jnp.tanh(c)
            hs_scr[pl.ds(k * 8 + j, 1), :] = h
        return (h, c)

    hN, cN = jax.lax.fori_loop(0, be // 8, chunk, (h_scr[...], c_scr[...]))
    h_scr[...] = hN
    c_scr[...] = cN
    out_ref[...] = jnp.dot(hs_scr[...], wlin_ref[...],
                           preferred_element_type=jnp.float32) + blin_ref[...]


def _edge_lstm(s_src, s_dst, attr_p, W_gcn, b_gcn, Wsrc_T, Wdst_T, Wattr_T,
               bias, Whh_T, Wlin_T, blin):
    e = s_src.shape[0]
    h = W_gcn.shape[1]
    lh = Whh_T.shape[0]
    g4 = 4 * lh
    ap = attr_p.shape[1]
    be = _pick_block(e)
    nb = e // be

    body = functools.partial(_lstm_body, be=be, lh=lh)
    out = pl.pallas_call(
        body,
        grid=(nb,),
        in_specs=[
            pl.BlockSpec((be, 1), lambda i: (i, 0)),
            pl.BlockSpec((be, 1), lambda i: (i, 0)),
            pl.BlockSpec((be, ap), lambda i: (i, 0)),
            pl.BlockSpec((1, h), lambda i: (0, 0)),
            pl.BlockSpec((1, h), lambda i: (0, 0)),
            pl.BlockSpec((h, g4), lambda i: (0, 0)),
            pl.BlockSpec((h, g4), lambda i: (0, 0)),
            pl.BlockSpec((ap, g4), lambda i: (0, 0)),
            pl.BlockSpec((1, g4), lambda i: (0, 0)),
            pl.BlockSpec((lh, g4), lambda i: (0, 0)),
            pl.BlockSpec((lh, 1), lambda i: (0, 0)),
            pl.BlockSpec((1, 1), lambda i: (0, 0)),
        ],
        out_specs=pl.BlockSpec((be, 1), lambda i: (i, 0)),
        out_shape=jax.ShapeDtypeStruct((e, 1), jnp.float32),
        scratch_shapes=[
            pltpu.VMEM((1, lh), jnp.float32),
            pltpu.VMEM((1, lh), jnp.float32),
            pltpu.VMEM((be, lh), jnp.float32),
            pltpu.VMEM((be, g4), jnp.float32),
        ],
    )(s_src, s_dst, attr_p, W_gcn, b_gcn, Wsrc_T, Wdst_T, Wattr_T, bias,
      Whh_T, Wlin_T, blin)
    return out


def kernel(x, edge_index, edge_attr, W_gcn, b_gcn, W_ih, W_hh, b_ih, b_hh,
           W_lin, b_lin):
    n = x.shape[0]
    e = edge_index.shape[1]
    h = W_gcn.shape[1]
    lh = W_hh.shape[1]
    a = edge_attr.shape[1]

    src = edge_index[0]
    dst = edge_index[1]

    xs = x[:, 0]
    deg = jnp.zeros((n,), jnp.float32).at[dst].add(1.0) + 1.0
    dinv = jax.lax.rsqrt(deg)
    av = xs * dinv
    spart = jnp.zeros((n,), jnp.float32).at[dst].add(av[src])
    s = dinv * spart + xs * dinv * dinv
    s_src = s[src][:, None]
    s_dst = s[dst][:, None]

    attr_p = jnp.pad(edge_attr, ((0, 0), (0, 8 - a)))
    Wsrc_T = W_ih[:, :h].T
    Wdst_T = W_ih[:, h:2 * h].T
    Wattr_T = jnp.pad(W_ih[:, 2 * h:].T, ((0, 8 - a), (0, 0)))
    bias = (b_ih + b_hh)[None, :]
    Whh_T = W_hh.T
    Wlin_T = W_lin.T
    blin = b_lin[None, :]

    out = _edge_lstm(s_src, s_dst, attr_p, W_gcn, b_gcn[None, :], Wsrc_T,
                     Wdst_T, Wattr_T, bias, Whh_T, Wlin_T, blin)
    return out.reshape(-1)
